# baseline (speedup 1.0000x reference)
import jax
import jax.numpy as jnp
from jax.experimental import pallas as pl

H = 256
NUM_LAYERS = 2
N_PEOPLE = 10000
N_LOC = 10000
E = 160000


def _lin(x, W, b):
    return x @ W.T + b


def _bn(x, g, b, eps=1e-5):
    mu = jnp.mean(x, axis=0)
    var = jnp.var(x, axis=0)
    return g * (x - mu) / jnp.sqrt(var + eps) + b


def _enc(x, W, b, g, beta):
    return jax.nn.relu(_bn(_lin(x, W, b), g, beta))


def _gat(x_src, x_dst, ei, W, b, a_src, a_dst, num_dst):
    h_src = x_src @ W.T
    h_dst = x_dst @ W.T
    s = ei[0]
    d = ei[1]
    alpha = jax.nn.leaky_relu((h_src * a_src).sum(-1)[s] + (h_dst * a_dst).sum(-1)[d], negative_slope=0.2)
    m = jax.ops.segment_max(alpha, d, num_segments=num_dst)
    e = jnp.exp(alpha - m[d])
    z = jax.ops.segment_sum(e, d, num_segments=num_dst)
    w = e / (z[d] + 1e-16)
    return jax.ops.segment_sum(h_src[s] * w[:, None], d, num_segments=num_dst) + b


def _head_plain(x, P, p):
    h = jax.nn.relu(_lin(x, P[p + '_W1'], P[p + '_b1']))
    h = jax.nn.relu(_lin(h, P[p + '_W2'], P[p + '_b2']))
    return _lin(h, P[p + '_W3'], P[p + '_b3'])


def _head_bn(x, P, p):
    h = jax.nn.relu(_bn(_lin(x, P[p + '_W1'], P[p + '_b1']), P[p + '_g1'], P[p + '_be1']))
    h = jax.nn.relu(_bn(_lin(h, P[p + '_W2'], P[p + '_b2']), P[p + '_g2'], P[p + '_be2']))
    return _lin(h, P[p + '_W3'], P[p + '_b3'])


def kernel(x_people, x_location, edge_attr_connected_to, edge_attr_visits, params, edge_index_lives_with, edge_index_connected_to, edge_index_visits, edge_index_visited_by):
    P = params
    ei_lw, ei_ct, ei_v, ei_vb = edge_index_lives_with, edge_index_connected_to, edge_index_visits, edge_index_visited_by
    p = _enc(x_people.reshape(-1, 1), P['enc_person_W'], P['enc_person_b'], P['enc_person_g'], P['enc_person_be'])
    l = _enc(x_location, P['enc_location_W'], P['enc_location_b'], P['enc_location_g'], P['enc_location_be'])
    for i in range(NUM_LAYERS):
        prev_p, prev_l = p, l
        out_p = (_gat(p, p, ei_lw, P['gat_W'][i, 0], P['gat_b'][i, 0], P['gat_as'][i, 0], P['gat_ad'][i, 0], N_PEOPLE)
                 + _gat(l, p, ei_vb, P['gat_W'][i, 3], P['gat_b'][i, 3], P['gat_as'][i, 3], P['gat_ad'][i, 3], N_PEOPLE)) * 0.5
        out_l = (_gat(l, l, ei_ct, P['gat_W'][i, 1], P['gat_b'][i, 1], P['gat_as'][i, 1], P['gat_ad'][i, 1], N_LOC)
                 + _gat(p, l, ei_v, P['gat_W'][i, 2], P['gat_b'][i, 2], P['gat_as'][i, 2], P['gat_ad'][i, 2], N_LOC)) * 0.5
        p = jax.nn.relu(_bn(out_p, P['bn_g'][i, 0], P['bn_b'][i, 0])) + prev_p
        l = jax.nn.relu(_bn(out_l, P['bn_g'][i, 1], P['bn_b'][i, 1])) + prev_l
    pair = jnp.concatenate([p, l], axis=1)
    edge_pred = jax.nn.sigmoid(_head_plain(pair, P, 'ep'))
    purpose_pred = _head_bn(pair, P, 'pp')
    time_pred = jax.nn.sigmoid(_head_bn(pair, P, 'tp')) * 24.0
    joint_pred = jax.nn.sigmoid(_head_bn(pair, P, 'jp'))
    return edge_pred, purpose_pred, time_pred, joint_pred, jnp.arange(N_PEOPLE), jnp.arange(N_LOC)


# speedup vs baseline: 5.2143x; 5.2143x over previous
import functools

import jax
import jax.numpy as jnp
from jax import lax
from jax.experimental import pallas as pl
from jax.experimental.pallas import tpu as pltpu
from jax.experimental.pallas import tpu_sc as plsc

H = 256
NUM_LAYERS = 2
N = 10000
E = 160000
HALF = N // 2
PAD = 5008
DUMP = 5000
EPT = E // 16
CH = 80
NCH = EPT // CH
RPT = PAD // 16


def _agg_body(s_hbm, d_hbm, asrc_hbm, adst_hbm, m_hbm, h_hbm, out_hbm,
              ebuf, dloc, zbuf, rowbuf, stmp, dtmp, atmp, btmp, mbuf,
              z_sh, out_sh):
    c = lax.axis_index("c")
    sub = lax.axis_index("s")
    lo = c * HALF
    zeros16 = jnp.zeros((16,), jnp.float32)
    base_e = sub * EPT

    pltpu.sync_copy(m_hbm, mbuf)

    def _zrow(r, _):
        for k in range(H // 16):
            rowbuf[r, pl.ds(k * 16, 16)] = zeros16
        return _
    lax.fori_loop(0, CH, _zrow, None)

    def _zz(i, _):
        zbuf[pl.ds(i * 16, 16)] = zeros16
        return _
    lax.fori_loop(0, PAD // 16, _zz, None)

    @pl.when(sub == 0)
    def _():
        pltpu.sync_copy(zbuf, z_sh)
    for b, sz in ((0, CH), (1, CH), (2, CH), (3, RPT - 3 * CH)):
        pltpu.sync_copy(rowbuf.at[pl.ds(0, sz)],
                        out_sh.at[pl.ds(sub * RPT + b * CH, sz)])

    plsc.subcore_barrier()

    mv = mbuf[...]

    def _escore(i, _):
        eb = base_e + i * CH
        pltpu.sync_copy(s_hbm.at[pl.ds(eb, CH)], stmp)
        pltpu.sync_copy(d_hbm.at[pl.ds(eb, CH)], dtmp)
        pltpu.sync_copy(asrc_hbm.at[stmp], atmp)
        pltpu.sync_copy(adst_hbm.at[dtmp], btmp)
        for u in range(5):
            base = i * CH + u * 16
            d16 = dtmp[pl.ds(u * 16, 16)]
            t = atmp[pl.ds(u * 16, 16)] + btmp[pl.ds(u * 16, 16)]
            alpha = jnp.where(t >= 0, t, t * jnp.float32(0.2))
            own = (d16 >= lo) & (d16 < lo + HALF)
            e = jnp.where(own, jnp.exp(alpha - mv), jnp.float32(0.0))
            ebuf[pl.ds(base, 16)] = e
            dloc[i, pl.ds(u * 16, 16)] = jnp.where(own, d16 - lo, jnp.int32(DUMP))
        return _
    lax.fori_loop(0, NCH, _escore, None)

    def _zscat(i, _):
        pltpu.sync_copy(ebuf.at[pl.ds(i * CH, CH)], z_sh.at[dloc.at[i]], add=True)
        return _
    lax.fori_loop(0, NCH, _zscat, None)

    plsc.subcore_barrier()

    pltpu.sync_copy(z_sh, zbuf)

    def _wcalc(i, _):
        for u in range(5):
            base = i * CH + u * 16
            dl = dloc[i, pl.ds(u * 16, 16)]
            zg = plsc.load_gather(zbuf, [dl])
            ebuf[pl.ds(base, 16)] = ebuf[pl.ds(base, 16)] / (zg + jnp.float32(1e-16))
        return _
    lax.fori_loop(0, NCH, _wcalc, None)

    def _chunk(i, _):
        pltpu.sync_copy(s_hbm.at[pl.ds(base_e + i * CH, CH)], stmp)
        pltpu.sync_copy(h_hbm.at[stmp], rowbuf)

        def _scale(r, _2):
            w = ebuf[pl.ds(i * CH + r, 16)][0]
            for k in range(H // 16):
                rowbuf[r, pl.ds(k * 16, 16)] = rowbuf[r, pl.ds(k * 16, 16)] * w
            return _2
        lax.fori_loop(0, CH, _scale, None)
        pltpu.sync_copy(rowbuf, out_sh.at[dloc.at[i]], add=True)
        return _
    lax.fori_loop(0, NCH, _chunk, None)

    plsc.subcore_barrier()

    @pl.when(sub < 15)
    def _():
        pltpu.sync_copy(out_sh.at[pl.ds(sub * RPT, RPT)],
                        out_hbm.at[pl.ds(lo + sub * RPT, RPT)])

    @pl.when(sub == 15)
    def _():
        pltpu.sync_copy(out_sh.at[pl.ds(15 * RPT, HALF - 15 * RPT)],
                        out_hbm.at[pl.ds(lo + 15 * RPT, HALF - 15 * RPT)])


_agg = pl.kernel(
    _agg_body,
    out_type=jax.ShapeDtypeStruct((N, H), jnp.float32),
    mesh=plsc.VectorSubcoreMesh(core_axis_name="c", subcore_axis_name="s"),
    compiler_params=pltpu.CompilerParams(needs_layout_passes=False,
                                         use_tc_tiling_on_sc=False),
    scratch_types=[
        pltpu.VMEM((EPT + 16,), jnp.float32),
        pltpu.VMEM((NCH, CH), jnp.int32),
        pltpu.VMEM((PAD,), jnp.float32),
        pltpu.VMEM((CH, H), jnp.float32),
        pltpu.VMEM((CH,), jnp.int32),
        pltpu.VMEM((CH,), jnp.int32),
        pltpu.VMEM((CH,), jnp.float32),
        pltpu.VMEM((CH,), jnp.float32),
        pltpu.VMEM((16,), jnp.float32),
        pltpu.VMEM_SHARED((PAD,), jnp.float32),
        pltpu.VMEM_SHARED((PAD, H), jnp.float32),
    ],
)


def _lin(x, W, b):
    return x @ W.T + b


def _bn(x, g, b, eps=1e-5):
    mu = jnp.mean(x, axis=0)
    var = jnp.var(x, axis=0)
    return g * (x - mu) / jnp.sqrt(var + eps) + b


def _enc(x, W, b, g, beta):
    return jax.nn.relu(_bn(_lin(x, W, b), g, beta))


def _gat_sc(x_src, x_dst, ei, W, b, a_src, a_dst):
    h_src = x_src @ W.T
    h_dst = x_dst @ W.T
    asrc = (h_src * a_src).sum(-1)
    adst = (h_dst * a_dst).sum(-1)
    t = jnp.max(asrc) + jnp.max(adst)
    m = jnp.where(t >= 0, t, t * 0.2)
    m16 = jnp.full((16,), m, jnp.float32)
    out = _agg(ei[0], ei[1], asrc, adst, m16, h_src)
    return out + b


def _head_plain(x, P, p):
    h = jax.nn.relu(_lin(x, P[p + '_W1'], P[p + '_b1']))
    h = jax.nn.relu(_lin(h, P[p + '_W2'], P[p + '_b2']))
    return _lin(h, P[p + '_W3'], P[p + '_b3'])


def _head_bn(x, P, p):
    h = jax.nn.relu(_bn(_lin(x, P[p + '_W1'], P[p + '_b1']), P[p + '_g1'], P[p + '_be1']))
    h = jax.nn.relu(_bn(_lin(h, P[p + '_W2'], P[p + '_b2']), P[p + '_g2'], P[p + '_be2']))
    return _lin(h, P[p + '_W3'], P[p + '_b3'])


def kernel(x_people, x_location, edge_attr_connected_to, edge_attr_visits, params, edge_index_lives_with, edge_index_connected_to, edge_index_visits, edge_index_visited_by):
    P = params
    p = _enc(x_people.reshape(-1, 1), P['enc_person_W'], P['enc_person_b'], P['enc_person_g'], P['enc_person_be'])
    l = _enc(x_location, P['enc_location_W'], P['enc_location_b'], P['enc_location_g'], P['enc_location_be'])
    for i in range(NUM_LAYERS):
        prev_p, prev_l = p, l
        out_p = (_gat_sc(p, p, edge_index_lives_with, P['gat_W'][i, 0], P['gat_b'][i, 0], P['gat_as'][i, 0], P['gat_ad'][i, 0])
                 + _gat_sc(l, p, edge_index_visited_by, P['gat_W'][i, 3], P['gat_b'][i, 3], P['gat_as'][i, 3], P['gat_ad'][i, 3])) * 0.5
        out_l = (_gat_sc(l, l, edge_index_connected_to, P['gat_W'][i, 1], P['gat_b'][i, 1], P['gat_as'][i, 1], P['gat_ad'][i, 1])
                 + _gat_sc(p, l, edge_index_visits, P['gat_W'][i, 2], P['gat_b'][i, 2], P['gat_as'][i, 2], P['gat_ad'][i, 2])) * 0.5
        p = jax.nn.relu(_bn(out_p, P['bn_g'][i, 0], P['bn_b'][i, 0])) + prev_p
        l = jax.nn.relu(_bn(out_l, P['bn_g'][i, 1], P['bn_b'][i, 1])) + prev_l
    pair = jnp.concatenate([p, l], axis=1)
    edge_pred = jax.nn.sigmoid(_head_plain(pair, P, 'ep'))
    purpose_pred = _head_bn(pair, P, 'pp')
    time_pred = jax.nn.sigmoid(_head_bn(pair, P, 'tp')) * 24.0
    joint_pred = jax.nn.sigmoid(_head_bn(pair, P, 'jp'))
    return edge_pred, purpose_pred, time_pred, joint_pred, jnp.arange(N), jnp.arange(N)
